# baseline (speedup 1.0000x reference)
import functools

import jax
import jax.numpy as jnp
from jax import lax
from jax.experimental import pallas as pl
from jax.experimental.pallas import tpu as pltpu
from jax.experimental.pallas import tpu_sc as plsc

NPTS = 10000
KNN = 32
D_E = 12
D_X = 14
D_F = 26
D_T = 32
D_H = 128

INTERPRET = False


def _embed_body(x_ref, pidf_ref, W1_ref, b1_ref, W2_ref, b2r_ref,
                Wd_ref, be1_ref, hp_ref, tab_ref, pb_ref):
    x = x_ref[...]
    h = jnp.maximum(jnp.dot(x, W1_ref[...]) + b1_ref[...], 0.0)
    Hm = jnp.dot(h, W2_ref[...]) + b2r_ref[...]
    nrm = jnp.sqrt(jnp.sum(Hm * Hm, axis=1, keepdims=True))
    Hn = Hm / (nrm + 1e-12)
    R = x.shape[0]
    hp_ref[...] = jnp.concatenate(
        [Hn, jnp.zeros((R, 4), jnp.float32)], axis=1)
    tab = jnp.concatenate(
        [Hn, x, pidf_ref[...], jnp.zeros((R, 5), jnp.float32)], axis=1)
    tab_ref[...] = tab
    pb_ref[...] = jnp.dot(tab, Wd_ref[...]) + be1_ref[...]


def _embed(x, pidf, W1, b1r, W2, b2r, Wd, be1r):
    R = 2000
    grid = NPTS // R
    return pl.pallas_call(
        _embed_body,
        grid=(grid,),
        in_specs=[
            pl.BlockSpec((R, D_X), lambda i: (i, 0)),
            pl.BlockSpec((R, 1), lambda i: (i, 0)),
            pl.BlockSpec((D_X, D_H), lambda i: (0, 0)),
            pl.BlockSpec((1, D_H), lambda i: (0, 0)),
            pl.BlockSpec((D_H, D_E), lambda i: (0, 0)),
            pl.BlockSpec((1, D_E), lambda i: (0, 0)),
            pl.BlockSpec((D_T, D_H), lambda i: (0, 0)),
            pl.BlockSpec((1, D_H), lambda i: (0, 0)),
        ],
        out_specs=[
            pl.BlockSpec((R, 16), lambda i: (i, 0)),
            pl.BlockSpec((R, D_T), lambda i: (i, 0)),
            pl.BlockSpec((R, D_H), lambda i: (i, 0)),
        ],
        out_shape=[
            jax.ShapeDtypeStruct((NPTS, 16), jnp.float32),
            jax.ShapeDtypeStruct((NPTS, D_T), jnp.float32),
            jax.ShapeDtypeStruct((NPTS, D_H), jnp.float32),
        ],
        interpret=INTERPRET,
    )(x, pidf, W1, b1r, W2, b2r, Wd, be1r)


_TKR = 400


def _topk_body(q_ref, kt_ref, sqr_ref, sqc_ref, nbr_ref, d2s_ref):
    i = pl.program_id(0)
    q = q_ref[...]
    kt = kt_ref[...]
    G = jax.lax.dot_general(q, kt, (((1,), (1,)), ((), ())),
                            preferred_element_type=jnp.float32)
    d2 = (sqr_ref[...] + sqc_ref[...]) - 2.0 * G
    col = jax.lax.broadcasted_iota(jnp.int32, (_TKR, NPTS), 1)
    rowg = i * _TKR + jax.lax.broadcasted_iota(jnp.int32, (_TKR, NPTS), 0)
    d2s_ref[...] = jnp.where(col == rowg, jnp.inf, d2)
    big = jnp.int32(2 ** 30)
    lane32 = jax.lax.broadcasted_iota(jnp.int32, (_TKR, KNN), 1)

    def body(k, carry):
        acc, mprev, iprev = carry
        cur = d2s_ref[...]
        dead = (cur < mprev) | ((cur == mprev) & (col <= iprev))
        eff = jnp.where(dead, jnp.inf, cur)
        m = jnp.min(eff, axis=1, keepdims=True)
        idx = jnp.min(jnp.where(eff == m, col, big),
                      axis=1, keepdims=True)
        return (jnp.where(lane32 == k, idx, acc), m, idx)

    acc0 = jnp.zeros((_TKR, KNN), jnp.int32)
    m0 = jnp.full((_TKR, 1), -jnp.inf, jnp.float32)
    i0 = jnp.full((_TKR, 1), -1, jnp.int32)
    nbr_ref[...] = jax.lax.fori_loop(0, KNN, body, (acc0, m0, i0))[0]


def _topk_pallas(hp, sqr, sqc):
    grid = NPTS // _TKR
    return pl.pallas_call(
        _topk_body,
        grid=(grid,),
        in_specs=[
            pl.BlockSpec((_TKR, 16), lambda i: (i, 0)),
            pl.BlockSpec((NPTS, 16), lambda i: (0, 0)),
            pl.BlockSpec((_TKR, 1), lambda i: (i, 0)),
            pl.BlockSpec((1, NPTS), lambda i: (0, 0)),
        ],
        out_specs=pl.BlockSpec((_TKR, KNN), lambda i: (i, 0)),
        out_shape=jax.ShapeDtypeStruct((NPTS, KNN), jnp.int32),
        scratch_shapes=[pltpu.VMEM((_TKR, NPTS), jnp.float32)],
        interpret=INTERPRET,
    )(hp, hp, sqr, sqc)


_GCH = 80
_NE = NPTS * KNN


def _gather_sc(tab, src):
    info = plsc.get_sparse_core_info()
    nc, ns = info.num_cores, info.num_subcores
    nw = nc * ns
    bpw = _NE // nw
    nch = bpw // _GCH
    mesh = plsc.VectorSubcoreMesh(core_axis_name="c", subcore_axis_name="s")

    @functools.partial(
        pl.kernel, mesh=mesh,
        compiler_params=pltpu.CompilerParams(use_tc_tiling_on_sc=False),
        out_type=jax.ShapeDtypeStruct((_NE, D_T), jnp.float32),
        scratch_types=[
            pltpu.VMEM((_GCH,), jnp.int32),
            pltpu.VMEM((_GCH, D_T), jnp.float32),
            pltpu.SemaphoreType.DMA,
        ],
    )
    def k(tab_hbm, idx_hbm, out_hbm, idx_v, rows_v, sem):
        wid = lax.axis_index("s") * nc + lax.axis_index("c")
        base = wid * bpw

        def body(j, carry):
            off = base + j * _GCH
            pltpu.sync_copy(idx_hbm.at[pl.ds(off, _GCH)], idx_v)
            pltpu.async_copy(tab_hbm.at[idx_v], rows_v, sem).wait()
            pltpu.sync_copy(rows_v, out_hbm.at[pl.ds(off, _GCH)])
            return carry

        lax.fori_loop(0, nch, body, 0)

    return k(tab, src)


def _edge_body(S_ref, tab_ref, pb_ref, Ws_ref, we2_ref, be2_ref,
               w_ref, y_ref):
    S = S_ref[...]
    B = pb_ref.shape[0]
    z = jax.lax.dot_general(S, Ws_ref[...],
                            (((1,), (0,)), ((), ())))
    z3 = z.reshape(B, KNN, D_H)
    z3 = jnp.maximum(z3 + pb_ref[...][:, None, :], 0.0)
    t = jnp.sum(z3 * we2_ref[...], axis=2) + be2_ref[...]
    w_ref[...] = 1.0 / (1.0 + jnp.exp(-t))
    ps = jax.lax.slice(S, (0, D_F), (B * KNN, D_F + 1)).reshape(B, KNN)
    pd = jax.lax.slice(tab_ref[...], (0, D_F), (B, D_F + 1))
    y_ref[...] = (ps == pd).astype(jnp.float32)


def _edge(S, tab, pb, Ws, we2b, be2b):
    B = 1000
    grid = NPTS // B
    return pl.pallas_call(
        _edge_body,
        grid=(grid,),
        in_specs=[
            pl.BlockSpec((B * KNN, D_T), lambda i: (i, 0)),
            pl.BlockSpec((B, D_T), lambda i: (i, 0)),
            pl.BlockSpec((B, D_H), lambda i: (i, 0)),
            pl.BlockSpec((D_T, D_H), lambda i: (0, 0)),
            pl.BlockSpec((1, 1, D_H), lambda i: (0, 0, 0)),
            pl.BlockSpec((1, 1), lambda i: (0, 0)),
        ],
        out_specs=[
            pl.BlockSpec((B, KNN), lambda i: (i, 0)),
            pl.BlockSpec((B, KNN), lambda i: (i, 0)),
        ],
        out_shape=[
            jax.ShapeDtypeStruct((NPTS, KNN), jnp.float32),
            jax.ShapeDtypeStruct((NPTS, KNN), jnp.float32),
        ],
        interpret=INTERPRET,
    )(S, tab, pb, Ws, we2b, be2b)


def kernel(x, particle_id, pt, W1, b1, W2, b2, We1, be1, we2, be2):
    pidf = particle_id.astype(jnp.float32).reshape(NPTS, 1)
    zpad = jnp.zeros((D_T - D_F, D_H), jnp.float32)
    Wsrc = jnp.concatenate([We1[:D_F], zpad], axis=0)
    Wdst = jnp.concatenate([We1[D_F:], zpad], axis=0)
    hp, tab, pb = _embed(x, pidf, W1, b1.reshape(1, D_H), W2,
                         b2.reshape(1, D_E), Wdst, be1.reshape(1, D_H))
    Hn = hp[:, :D_E]
    sq = jnp.sum(Hn * Hn, axis=1)
    nbr = _topk_pallas(hp, sq.reshape(NPTS, 1), sq.reshape(1, NPTS))
    src = nbr.reshape(-1)
    S = _gather_sc(tab, src)
    w, y = _edge(S, tab, pb, Wsrc,
                 we2.reshape(1, 1, D_H), be2.reshape(1, 1))
    dst = jnp.repeat(jnp.arange(NPTS, dtype=nbr.dtype), KNN)
    edge_index = jnp.stack([src, dst], axis=0)
    return (w.reshape(-1), y.reshape(-1), edge_index)


# speedup vs baseline: 1.3488x; 1.3488x over previous
import functools

import jax
import jax.numpy as jnp
from jax import lax
from jax.experimental import pallas as pl
from jax.experimental.pallas import tpu as pltpu
from jax.experimental.pallas import tpu_sc as plsc

NPTS = 10000
KNN = 32
D_E = 12
D_X = 14
D_F = 26
D_T = 32
D_H = 128


def _embed_body(x_ref, pidf_ref, W1_ref, b1_ref, W2_ref, b2r_ref,
                Wd_ref, be1_ref, hp_ref, tab_ref, pb_ref):
    x = x_ref[...]
    h = jnp.maximum(jnp.dot(x, W1_ref[...]) + b1_ref[...], 0.0)
    Hm = jnp.dot(h, W2_ref[...]) + b2r_ref[...]
    nrm = jnp.sqrt(jnp.sum(Hm * Hm, axis=1, keepdims=True))
    Hn = Hm / (nrm + 1e-12)
    R = x.shape[0]
    hp_ref[...] = jnp.concatenate(
        [Hn, jnp.zeros((R, 4), jnp.float32)], axis=1)
    tab = jnp.concatenate(
        [Hn, x, pidf_ref[...], jnp.zeros((R, 5), jnp.float32)], axis=1)
    tab_ref[...] = tab
    pb_ref[...] = jnp.dot(tab, Wd_ref[...]) + be1_ref[...]


def _embed(x, pidf, W1, b1r, W2, b2r, Wd, be1r):
    R = 2000
    grid = NPTS // R
    return pl.pallas_call(
        _embed_body,
        grid=(grid,),
        in_specs=[
            pl.BlockSpec((R, D_X), lambda i: (i, 0)),
            pl.BlockSpec((R, 1), lambda i: (i, 0)),
            pl.BlockSpec((D_X, D_H), lambda i: (0, 0)),
            pl.BlockSpec((1, D_H), lambda i: (0, 0)),
            pl.BlockSpec((D_H, D_E), lambda i: (0, 0)),
            pl.BlockSpec((1, D_E), lambda i: (0, 0)),
            pl.BlockSpec((D_T, D_H), lambda i: (0, 0)),
            pl.BlockSpec((1, D_H), lambda i: (0, 0)),
        ],
        out_specs=[
            pl.BlockSpec((R, 16), lambda i: (i, 0)),
            pl.BlockSpec((R, D_T), lambda i: (i, 0)),
            pl.BlockSpec((R, D_H), lambda i: (i, 0)),
        ],
        out_shape=[
            jax.ShapeDtypeStruct((NPTS, 16), jnp.float32),
            jax.ShapeDtypeStruct((NPTS, D_T), jnp.float32),
            jax.ShapeDtypeStruct((NPTS, D_H), jnp.float32),
        ],
    )(x, pidf, W1, b1r, W2, b2r, Wd, be1r)


_TKR = 400


def _topk_body(q_ref, kt_ref, sqr_ref, sqc_ref, nbr_ref, d2s_ref):
    i = pl.program_id(0)
    q = q_ref[...]
    kt = kt_ref[...]
    G = jax.lax.dot_general(q, kt, (((1,), (1,)), ((), ())),
                            preferred_element_type=jnp.float32)
    d2 = (sqr_ref[...] + sqc_ref[...]) - 2.0 * G
    col = jax.lax.broadcasted_iota(jnp.int32, (_TKR, NPTS), 1)
    rowg = i * _TKR + jax.lax.broadcasted_iota(jnp.int32, (_TKR, NPTS), 0)
    d2s_ref[...] = jnp.where(col == rowg, jnp.inf, d2)
    big = jnp.int32(2 ** 30)
    lane32 = jax.lax.broadcasted_iota(jnp.int32, (_TKR, KNN), 1)

    def body(k, acc):
        cur = d2s_ref[...]
        m = jnp.min(cur, axis=1, keepdims=True)
        idx = jnp.min(jnp.where(cur == m, col, big),
                      axis=1, keepdims=True)
        d2s_ref[...] = jnp.where(col == idx, jnp.inf, cur)
        return jnp.where(lane32 == k, idx, acc)

    acc0 = jnp.zeros((_TKR, KNN), jnp.int32)
    nbr_ref[...] = jax.lax.fori_loop(0, KNN, body, acc0)


def _topk_pallas(hp, sqr, sqc):
    grid = NPTS // _TKR
    return pl.pallas_call(
        _topk_body,
        grid=(grid,),
        in_specs=[
            pl.BlockSpec((_TKR, 16), lambda i: (i, 0)),
            pl.BlockSpec((NPTS, 16), lambda i: (0, 0)),
            pl.BlockSpec((_TKR, 1), lambda i: (i, 0)),
            pl.BlockSpec((1, NPTS), lambda i: (0, 0)),
        ],
        out_specs=pl.BlockSpec((_TKR, KNN), lambda i: (i, 0)),
        out_shape=jax.ShapeDtypeStruct((NPTS, KNN), jnp.int32),
        scratch_shapes=[pltpu.VMEM((_TKR, NPTS), jnp.float32)],
    )(hp, hp, sqr, sqc)


_GCH = 80
_NE = NPTS * KNN


def _gather_sc(tab, src):
    info = plsc.get_sparse_core_info()
    nc, ns = info.num_cores, info.num_subcores
    nw = nc * ns
    bpw = _NE // nw
    nch = bpw // _GCH
    mesh = plsc.VectorSubcoreMesh(core_axis_name="c", subcore_axis_name="s")

    @functools.partial(
        pl.kernel, mesh=mesh,
        compiler_params=pltpu.CompilerParams(use_tc_tiling_on_sc=False),
        out_type=jax.ShapeDtypeStruct((_NE, D_T), jnp.float32),
        scratch_types=[
            pltpu.VMEM((_GCH,), jnp.int32),
            pltpu.VMEM((_GCH, D_T), jnp.float32),
            pltpu.SemaphoreType.DMA,
        ],
    )
    def k(tab_hbm, idx_hbm, out_hbm, idx_v, rows_v, sem):
        wid = lax.axis_index("s") * nc + lax.axis_index("c")
        base = wid * bpw

        def body(j, carry):
            off = base + j * _GCH
            pltpu.sync_copy(idx_hbm.at[pl.ds(off, _GCH)], idx_v)
            pltpu.async_copy(tab_hbm.at[idx_v], rows_v, sem).wait()
            pltpu.sync_copy(rows_v, out_hbm.at[pl.ds(off, _GCH)])
            return carry

        lax.fori_loop(0, nch, body, 0)

    return k(tab, src)


def _edge_body(S_ref, tab_ref, pb_ref, Ws_ref, we2_ref, be2_ref,
               w_ref, y_ref):
    S = S_ref[...]
    B = pb_ref.shape[0]
    z = jax.lax.dot_general(S, Ws_ref[...],
                            (((1,), (0,)), ((), ())))
    z3 = z.reshape(B, KNN, D_H)
    z3 = jnp.maximum(z3 + pb_ref[...][:, None, :], 0.0)
    t = jnp.sum(z3 * we2_ref[...], axis=2) + be2_ref[...]
    w_ref[...] = 1.0 / (1.0 + jnp.exp(-t))
    ps = jax.lax.slice(S, (0, D_F), (B * KNN, D_F + 1)).reshape(B, KNN)
    pd = jax.lax.slice(tab_ref[...], (0, D_F), (B, D_F + 1))
    y_ref[...] = (ps == pd).astype(jnp.float32)


def _edge(S, tab, pb, Ws, we2b, be2b):
    B = 1000
    grid = NPTS // B
    return pl.pallas_call(
        _edge_body,
        grid=(grid,),
        in_specs=[
            pl.BlockSpec((B * KNN, D_T), lambda i: (i, 0)),
            pl.BlockSpec((B, D_T), lambda i: (i, 0)),
            pl.BlockSpec((B, D_H), lambda i: (i, 0)),
            pl.BlockSpec((D_T, D_H), lambda i: (0, 0)),
            pl.BlockSpec((1, 1, D_H), lambda i: (0, 0, 0)),
            pl.BlockSpec((1, 1), lambda i: (0, 0)),
        ],
        out_specs=[
            pl.BlockSpec((B, KNN), lambda i: (i, 0)),
            pl.BlockSpec((B, KNN), lambda i: (i, 0)),
        ],
        out_shape=[
            jax.ShapeDtypeStruct((NPTS, KNN), jnp.float32),
            jax.ShapeDtypeStruct((NPTS, KNN), jnp.float32),
        ],
    )(S, tab, pb, Ws, we2b, be2b)


def kernel(x, particle_id, pt, W1, b1, W2, b2, We1, be1, we2, be2):
    pidf = particle_id.astype(jnp.float32).reshape(NPTS, 1)
    zpad = jnp.zeros((D_T - D_F, D_H), jnp.float32)
    Wsrc = jnp.concatenate([We1[:D_F], zpad], axis=0)
    Wdst = jnp.concatenate([We1[D_F:], zpad], axis=0)
    hp, tab, pb = _embed(x, pidf, W1, b1.reshape(1, D_H), W2,
                         b2.reshape(1, D_E), Wdst, be1.reshape(1, D_H))
    Hn = hp[:, :D_E]
    sq = jnp.sum(Hn * Hn, axis=1)
    nbr = _topk_pallas(hp, sq.reshape(NPTS, 1), sq.reshape(1, NPTS))
    src = nbr.reshape(-1)
    S = _gather_sc(tab, src)
    w, y = _edge(S, tab, pb, Wsrc,
                 we2.reshape(1, 1, D_H), be2.reshape(1, 1))
    dst = jnp.repeat(jnp.arange(NPTS, dtype=nbr.dtype), KNN)
    edge_index = jnp.stack([src, dst], axis=0)
    return (w.reshape(-1), y.reshape(-1), edge_index)
